# baseline (speedup 1.0000x reference)
import functools

import jax
import jax.numpy as jnp
from jax import lax
from jax.experimental import pallas as pl
from jax.experimental.pallas import tpu as pltpu
from jax.experimental.pallas import tpu_sc as plsc

_VOCAB = 100000
_D = 768
_SEQ = 2048

_NC = 2
_NS = 16
_NW = _NC * _NS
_B_PER_W = _SEQ // _NW

_VB = 1024


def _sc_gather_build():
    mesh = plsc.VectorSubcoreMesh(core_axis_name="c", subcore_axis_name="s")

    @functools.partial(
        pl.kernel,
        mesh=mesh,
        out_type=jax.ShapeDtypeStruct((_SEQ, _D), jnp.float32),
        scratch_types=[
            pltpu.VMEM((_B_PER_W,), jnp.int32),
            pltpu.VMEM((_B_PER_W, _D), jnp.float32),
            pltpu.SemaphoreType.DMA,
        ],
    )
    def gather_k(ids_hbm, table_hbm, out_hbm, idx_v, rows_v, sem):
        wid = lax.axis_index("s") * _NC + lax.axis_index("c")
        base = wid * _B_PER_W
        pltpu.sync_copy(ids_hbm.at[pl.ds(base, _B_PER_W)], idx_v)
        pltpu.async_copy(table_hbm.at[idx_v], rows_v, sem).wait()
        pltpu.sync_copy(rows_v, out_hbm.at[pl.ds(base, _B_PER_W)])

    return gather_k


_sc_gather = _sc_gather_build()


def _mm_body(x_ref, tab_ref, out_ref):
    tab = tab_ref[...].astype(jnp.bfloat16)
    out_ref[...] = lax.dot_general(
        x_ref[...],
        tab,
        dimension_numbers=(((1,), (1,)), ((), ())),
        preferred_element_type=jnp.float32,
    )


def _tc_matmul(x_bf16, table):
    grid = (pl.cdiv(_VOCAB, _VB),)
    return pl.pallas_call(
        _mm_body,
        grid=grid,
        in_specs=[
            pl.BlockSpec((_SEQ, _D), lambda i: (0, 0)),
            pl.BlockSpec((_VB, _D), lambda i: (i, 0)),
        ],
        out_specs=pl.BlockSpec((_SEQ, _VB), lambda i: (0, i)),
        out_shape=jax.ShapeDtypeStruct((_SEQ, _VOCAB), jnp.float32),
    )(x_bf16, table)


def kernel(input_ids, table):
    ids = input_ids.reshape(-1).astype(jnp.int32)
    x = _sc_gather(ids, table)
    logits = _tc_matmul(x.astype(jnp.bfloat16), table)
    return logits.reshape(1, _SEQ, _VOCAB)
